# baseline (speedup 1.0000x reference)
import functools

import jax
import jax.numpy as jnp
from jax import lax
from jax.experimental import pallas as pl
from jax.experimental.pallas import tpu as pltpu
from jax.experimental.pallas import tpu_sc as plsc

B, N = 128, 100000

TC_N = 80800
CHS = 3232
NCS = TC_N // CHS
SLS = CHS // 8

SC_START = TC_N
SC_N = N - TC_N
NW = 32
RPW = SC_N // NW
SLAB = 120
NSLAB = RPW // SLAB

CH = 4000
NC = N // CH
SL = CH // 8


def _stats_body(lg_ref, mk_ref, am_ref, as_ref, accm_ref, accs_ref):
    c = pl.program_id(0)
    NEG = jnp.float32(-1e9)

    @pl.when(c == 0)
    def _():
        accm_ref[...] = jnp.full((8, B), NEG, jnp.float32)
        accs_ref[...] = jnp.zeros((8, B), jnp.float32)

    mkf = mk_ref[...].astype(jnp.float32)
    ml3 = (lg_ref[...] * mkf + (mkf - 1.0) * (-NEG)).reshape(SLS, 8, B)
    cm = jnp.max(ml3, axis=0)
    am = accm_ref[...]
    nm = jnp.maximum(am, cm)
    cs = jnp.sum(jnp.exp(ml3 - nm[None]), axis=0)
    ns = accs_ref[...] * jnp.exp(am - nm) + cs
    accm_ref[...] = nm
    accs_ref[...] = ns

    @pl.when(c == NCS - 1)
    def _():
        am_ref[...] = nm
        as_ref[...] = ns


def _sc_stats(lg_hbm, mkf_hbm, m_out, s_out, lg_v, mk_v, st_v):
    wid = lax.axis_index("s") * 2 + lax.axis_index("c")
    base = SC_START + wid * RPW
    NEG = jnp.float32(-1e9)

    def slab_loop(pass2, carry0):
        def body(t, carry):
            pltpu.sync_copy(lg_hbm.at[pl.ds(base + t * SLAB, SLAB)], lg_v)
            pltpu.sync_copy(mkf_hbm.at[pl.ds(wid * RPW + t * SLAB, SLAB)], mk_v)

            def row(r, c):
                out = []
                for k in range(8):
                    x = lg_v[r, pl.ds(k * 16, 16)]
                    mf = mk_v[r, pl.ds(k * 16, 16)]
                    ml = x * mf + (mf - 1.0) * jnp.float32(1e9)
                    if pass2:
                        out.append(c[k] + jnp.exp(ml - c[8 + k]))
                    else:
                        out.append(jnp.maximum(c[k], ml))
                return tuple(out) + tuple(c[8:])

            return lax.fori_loop(0, SLAB, row, carry)

        return lax.fori_loop(0, NSLAB, body, carry0)

    negs = tuple(jnp.full((16,), NEG, jnp.float32) for _ in range(8))
    mres = slab_loop(False, negs + negs)[:8]
    zeros = tuple(jnp.zeros((16,), jnp.float32) for _ in range(8))
    sres = slab_loop(True, zeros + mres)[:8]

    for k in range(8):
        st_v[0, pl.ds(k * 16, 16)] = mres[k]
        st_v[1, pl.ds(k * 16, 16)] = sres[k]
    pltpu.sync_copy(st_v.at[0], m_out.at[wid])
    pltpu.sync_copy(st_v.at[1], s_out.at[wid])


def _argmax_body(lg_ref, mk_ref, gm_ref, am_ref, as_ref, scm_ref, scs_ref,
                 act_ref, lp_ref, m_scr, ic_scr, vm_ref, ix_ref, gl_ref):
    c = pl.program_id(0)
    NEG = jnp.float32(-1e9)
    BIG = jnp.int32(2 ** 30)

    @pl.when(c == 0)
    def _():
        am = am_ref[...]
        sc_m = scm_ref[...]
        m_b = jnp.maximum(jnp.max(am, axis=0, keepdims=True),
                          jnp.max(sc_m, axis=0, keepdims=True))
        ssum = (jnp.sum(as_ref[...] * jnp.exp(am - m_b), axis=0, keepdims=True)
                + jnp.sum(scs_ref[...] * jnp.exp(sc_m - m_b), axis=0,
                          keepdims=True))
        m_scr[...] = m_b
        ic_scr[...] = jnp.where(m_b > jnp.float32(-0.5e9),
                                1.0 / (ssum * (1.0 + jnp.float32(1e-8))), 0.0)
        vm_ref[...] = jnp.full((8, B), -jnp.inf, jnp.float32)
        ix_ref[...] = jnp.zeros((8, B), jnp.int32)
        gl_ref[...] = jnp.zeros((8, B), jnp.float32)

    gm3 = gm_ref[...].reshape(SL, 8, B)
    mkf = mk_ref[...].astype(jnp.float32)
    ml3 = (lg_ref[...] * mkf + (mkf - 1.0) * (-NEG)).reshape(SL, 8, B)
    m_b = m_scr[...][None]
    ic = ic_scr[...][None]
    v3 = jnp.log(jnp.exp(ml3 - m_b) * ic + jnp.float32(1e-9)) + gm3

    cmv = jnp.max(v3, axis=0)
    ci = jnp.argmax(v3, axis=0).astype(jnp.int32)
    i3 = jax.lax.broadcasted_iota(jnp.int32, (SL, 8, B), 0)
    cg = jnp.sum(jnp.where(i3 == ci[None], gm3, 0.0), axis=0)

    sub = jax.lax.broadcasted_iota(jnp.int32, (8, B), 0)
    cr = c * CH + ci * 8 + sub

    vm = vm_ref[...]
    upd = cmv > vm
    nvm = jnp.where(upd, cmv, vm)
    nix = jnp.where(upd, cr, ix_ref[...])
    ngl = jnp.where(upd, cg, gl_ref[...])
    vm_ref[...] = nvm
    ix_ref[...] = nix
    gl_ref[...] = ngl

    @pl.when(c == NC - 1)
    def _():
        vmax = jnp.max(nvm, axis=0, keepdims=True)
        cand = jnp.min(jnp.where(nvm == vmax, nix, BIG), axis=0, keepdims=True)
        g_at = jnp.sum(jnp.where(nix == cand, ngl, 0.0), axis=0, keepdims=True)
        act_ref[...] = cand
        lp_ref[...] = vmax - g_at


def _sc_stats_call(lgt, mkf_sc):
    mesh = plsc.VectorSubcoreMesh(core_axis_name="c", subcore_axis_name="s")
    kfn = functools.partial(
        pl.kernel,
        mesh=mesh,
        out_type=[
            jax.ShapeDtypeStruct((NW, B), jnp.float32),
            jax.ShapeDtypeStruct((NW, B), jnp.float32),
        ],
        scratch_types=[
            pltpu.VMEM((SLAB, B), jnp.float32),
            pltpu.VMEM((SLAB, B), jnp.float32),
            pltpu.VMEM((2, B), jnp.float32),
        ],
    )(_sc_stats)
    return kfn(lgt, mkf_sc)


def kernel(logits, mask, gumbel):
    lgt = logits.T
    gmt = gumbel.T
    mkt = mask.T.astype(jnp.uint8)
    mkf_sc = mask.T[SC_START:].astype(jnp.float32)

    am, asum = pl.pallas_call(
        _stats_body,
        grid=(NCS,),
        in_specs=[
            pl.BlockSpec((CHS, B), lambda c: (c, 0)),
            pl.BlockSpec((CHS, B), lambda c: (c, 0)),
        ],
        out_specs=[
            pl.BlockSpec((8, B), lambda c: (0, 0)),
            pl.BlockSpec((8, B), lambda c: (0, 0)),
        ],
        out_shape=[
            jax.ShapeDtypeStruct((8, B), jnp.float32),
            jax.ShapeDtypeStruct((8, B), jnp.float32),
        ],
        scratch_shapes=[
            pltpu.VMEM((8, B), jnp.float32),
            pltpu.VMEM((8, B), jnp.float32),
        ],
    )(lgt, mkt)

    scm, scs = _sc_stats_call(lgt, mkf_sc)

    acts, lps = pl.pallas_call(
        _argmax_body,
        grid=(NC,),
        in_specs=[
            pl.BlockSpec((CH, B), lambda c: (c, 0)),
            pl.BlockSpec((CH, B), lambda c: (c, 0)),
            pl.BlockSpec((CH, B), lambda c: (c, 0)),
            pl.BlockSpec((8, B), lambda c: (0, 0)),
            pl.BlockSpec((8, B), lambda c: (0, 0)),
            pl.BlockSpec((NW, B), lambda c: (0, 0)),
            pl.BlockSpec((NW, B), lambda c: (0, 0)),
        ],
        out_specs=[
            pl.BlockSpec((1, B), lambda c: (0, 0)),
            pl.BlockSpec((1, B), lambda c: (0, 0)),
        ],
        out_shape=[
            jax.ShapeDtypeStruct((1, B), jnp.int32),
            jax.ShapeDtypeStruct((1, B), jnp.float32),
        ],
        scratch_shapes=[
            pltpu.VMEM((1, B), jnp.float32),
            pltpu.VMEM((1, B), jnp.float32),
            pltpu.VMEM((8, B), jnp.float32),
            pltpu.VMEM((8, B), jnp.int32),
            pltpu.VMEM((8, B), jnp.float32),
        ],
    )(lgt, mkt, gmt, am, asum, scm, scs)

    return acts.reshape(B), lps.reshape(B)


# speedup vs baseline: 1.0557x; 1.0557x over previous
import functools

import jax
import jax.numpy as jnp
from jax import lax
from jax.experimental import pallas as pl
from jax.experimental.pallas import tpu as pltpu
from jax.experimental.pallas import tpu_sc as plsc

B, N = 128, 100000

TC_N = 89760
CHS = 5280
NCS = TC_N // CHS
SLS = CHS // 8

SC_START = TC_N
SC_N = N - TC_N
NW = 32
RPW = SC_N // NW
SLAB = 160
NSLAB = RPW // SLAB

CH = 4000
NC = N // CH
SL = CH // 8


def _stats_body(lg_ref, mk_ref, am_ref, as_ref, accm_ref, accs_ref):
    c = pl.program_id(0)
    NEG = jnp.float32(-1e9)

    @pl.when(c == 0)
    def _():
        accm_ref[...] = jnp.full((8, B), NEG, jnp.float32)
        accs_ref[...] = jnp.zeros((8, B), jnp.float32)

    mkf = mk_ref[...].astype(jnp.float32)
    ml3 = (lg_ref[...] * mkf + (mkf - 1.0) * (-NEG)).reshape(SLS, 8, B)
    cm = jnp.max(ml3, axis=0)
    am = accm_ref[...]
    nm = jnp.maximum(am, cm)
    cs = jnp.sum(jnp.exp(ml3 - nm[None]), axis=0)
    ns = accs_ref[...] * jnp.exp(am - nm) + cs
    accm_ref[...] = nm
    accs_ref[...] = ns

    @pl.when(c == NCS - 1)
    def _():
        am_ref[...] = nm
        as_ref[...] = ns


def _sc_stats(lg_hbm, mkf_hbm, m_out, s_out, lg_v, mk_v, st_v):
    wid = lax.axis_index("s") * 2 + lax.axis_index("c")
    base = SC_START + wid * RPW
    NEG = jnp.float32(-1e9)

    def slab_loop(pass2, carry0):
        def body(t, carry):
            pltpu.sync_copy(lg_hbm.at[pl.ds(base + t * SLAB, SLAB)], lg_v)
            pltpu.sync_copy(mkf_hbm.at[pl.ds(wid * RPW + t * SLAB, SLAB)], mk_v)

            def row(r, c):
                out = []
                for k in range(8):
                    x = lg_v[r, pl.ds(k * 16, 16)]
                    mf = mk_v[r, pl.ds(k * 16, 16)]
                    ml = x * mf + (mf - 1.0) * jnp.float32(1e9)
                    if pass2:
                        out.append(c[k] + jnp.exp(ml - c[8 + k]))
                    else:
                        out.append(jnp.maximum(c[k], ml))
                return tuple(out) + tuple(c[8:])

            return lax.fori_loop(0, SLAB, row, carry)

        return lax.fori_loop(0, NSLAB, body, carry0)

    negs = tuple(jnp.full((16,), NEG, jnp.float32) for _ in range(8))
    mres = slab_loop(False, negs + negs)[:8]
    zeros = tuple(jnp.zeros((16,), jnp.float32) for _ in range(8))
    sres = slab_loop(True, zeros + mres)[:8]

    for k in range(8):
        st_v[0, pl.ds(k * 16, 16)] = mres[k]
        st_v[1, pl.ds(k * 16, 16)] = sres[k]
    pltpu.sync_copy(st_v.at[0], m_out.at[wid])
    pltpu.sync_copy(st_v.at[1], s_out.at[wid])


def _argmax_body(lg_ref, mk_ref, gm_ref, am_ref, as_ref, scm_ref, scs_ref,
                 act_ref, lp_ref, m_scr, ic_scr, vm_ref, ix_ref, gl_ref):
    c = pl.program_id(0)
    NEG = jnp.float32(-1e9)
    BIG = jnp.int32(2 ** 30)

    @pl.when(c == 0)
    def _():
        am = am_ref[...]
        sc_m = scm_ref[...]
        m_b = jnp.maximum(jnp.max(am, axis=0, keepdims=True),
                          jnp.max(sc_m, axis=0, keepdims=True))
        ssum = (jnp.sum(as_ref[...] * jnp.exp(am - m_b), axis=0, keepdims=True)
                + jnp.sum(scs_ref[...] * jnp.exp(sc_m - m_b), axis=0,
                          keepdims=True))
        m_scr[...] = m_b
        ic_scr[...] = jnp.where(m_b > jnp.float32(-0.5e9),
                                1.0 / (ssum * (1.0 + jnp.float32(1e-8))), 0.0)
        vm_ref[...] = jnp.full((8, B), -jnp.inf, jnp.float32)
        ix_ref[...] = jnp.zeros((8, B), jnp.int32)
        gl_ref[...] = jnp.zeros((8, B), jnp.float32)

    gm3 = gm_ref[...].reshape(SL, 8, B)
    mkf = mk_ref[...].astype(jnp.float32)
    ml3 = (lg_ref[...] * mkf + (mkf - 1.0) * (-NEG)).reshape(SL, 8, B)
    m_b = m_scr[...][None]
    ic = ic_scr[...][None]
    v3 = jnp.log(jnp.exp(ml3 - m_b) * ic + jnp.float32(1e-9)) + gm3

    cmv = jnp.max(v3, axis=0)
    ci = jnp.argmax(v3, axis=0).astype(jnp.int32)
    i3 = jax.lax.broadcasted_iota(jnp.int32, (SL, 8, B), 0)
    cg = jnp.sum(jnp.where(i3 == ci[None], gm3, 0.0), axis=0)

    sub = jax.lax.broadcasted_iota(jnp.int32, (8, B), 0)
    cr = c * CH + ci * 8 + sub

    vm = vm_ref[...]
    upd = cmv > vm
    nvm = jnp.where(upd, cmv, vm)
    nix = jnp.where(upd, cr, ix_ref[...])
    ngl = jnp.where(upd, cg, gl_ref[...])
    vm_ref[...] = nvm
    ix_ref[...] = nix
    gl_ref[...] = ngl

    @pl.when(c == NC - 1)
    def _():
        vmax = jnp.max(nvm, axis=0, keepdims=True)
        cand = jnp.min(jnp.where(nvm == vmax, nix, BIG), axis=0, keepdims=True)
        g_at = jnp.sum(jnp.where(nix == cand, ngl, 0.0), axis=0, keepdims=True)
        act_ref[...] = cand
        lp_ref[...] = vmax - g_at


def _sc_stats_call(lgt, mkf_sc):
    mesh = plsc.VectorSubcoreMesh(core_axis_name="c", subcore_axis_name="s")
    kfn = functools.partial(
        pl.kernel,
        mesh=mesh,
        out_type=[
            jax.ShapeDtypeStruct((NW, B), jnp.float32),
            jax.ShapeDtypeStruct((NW, B), jnp.float32),
        ],
        scratch_types=[
            pltpu.VMEM((SLAB, B), jnp.float32),
            pltpu.VMEM((SLAB, B), jnp.float32),
            pltpu.VMEM((2, B), jnp.float32),
        ],
    )(_sc_stats)
    return kfn(lgt, mkf_sc)


def kernel(logits, mask, gumbel):
    lgt = logits.T
    gmt = gumbel.T
    mkt = mask.T.astype(jnp.uint8)
    mkf_sc = mask.T[SC_START:].astype(jnp.float32)

    am, asum = pl.pallas_call(
        _stats_body,
        grid=(NCS,),
        in_specs=[
            pl.BlockSpec((CHS, B), lambda c: (c, 0)),
            pl.BlockSpec((CHS, B), lambda c: (c, 0)),
        ],
        out_specs=[
            pl.BlockSpec((8, B), lambda c: (0, 0)),
            pl.BlockSpec((8, B), lambda c: (0, 0)),
        ],
        out_shape=[
            jax.ShapeDtypeStruct((8, B), jnp.float32),
            jax.ShapeDtypeStruct((8, B), jnp.float32),
        ],
        scratch_shapes=[
            pltpu.VMEM((8, B), jnp.float32),
            pltpu.VMEM((8, B), jnp.float32),
        ],
    )(lgt, mkt)

    scm, scs = _sc_stats_call(lgt, mkf_sc)

    acts, lps = pl.pallas_call(
        _argmax_body,
        grid=(NC,),
        in_specs=[
            pl.BlockSpec((CH, B), lambda c: (c, 0)),
            pl.BlockSpec((CH, B), lambda c: (c, 0)),
            pl.BlockSpec((CH, B), lambda c: (c, 0)),
            pl.BlockSpec((8, B), lambda c: (0, 0)),
            pl.BlockSpec((8, B), lambda c: (0, 0)),
            pl.BlockSpec((NW, B), lambda c: (0, 0)),
            pl.BlockSpec((NW, B), lambda c: (0, 0)),
        ],
        out_specs=[
            pl.BlockSpec((1, B), lambda c: (0, 0)),
            pl.BlockSpec((1, B), lambda c: (0, 0)),
        ],
        out_shape=[
            jax.ShapeDtypeStruct((1, B), jnp.int32),
            jax.ShapeDtypeStruct((1, B), jnp.float32),
        ],
        scratch_shapes=[
            pltpu.VMEM((1, B), jnp.float32),
            pltpu.VMEM((1, B), jnp.float32),
            pltpu.VMEM((8, B), jnp.float32),
            pltpu.VMEM((8, B), jnp.int32),
            pltpu.VMEM((8, B), jnp.float32),
        ],
    )(lgt, mkt, gmt, am, asum, scm, scs)

    return acts.reshape(B), lps.reshape(B)
